# baseline (speedup 1.0000x reference)
import functools

import jax
import jax.numpy as jnp
from jax import lax
from jax.experimental import pallas as pl
from jax.experimental.pallas import tpu as pltpu
from jax.experimental.pallas import tpu_sc as plsc

E = 768
LANES = 16
CHUNK = 16


def _make_sc_kernel(B, L, num_workers, rows_per_w, nchunk):
    mesh = plsc.VectorSubcoreMesh(core_axis_name="c", subcore_axis_name="s")
    info = plsc.get_sparse_core_info()
    nc = info.num_cores

    XBUF = 4
    GBUF = 2

    @functools.partial(
        pl.kernel,
        mesh=mesh,
        out_type=jax.ShapeDtypeStruct((B, L, E), jnp.float32),
        scratch_types=[
            pltpu.VMEM((rows_per_w,), jnp.int32),
            pltpu.VMEM((GBUF, CHUNK, E), jnp.float32),
            pltpu.VMEM((XBUF, CHUNK, E), jnp.float32),
        ] + [pltpu.SemaphoreType.DMA] * (GBUF + 2 * XBUF),
    )
    def sc_embed(table_hbm, idx_hbm, x_hbm, out_hbm,
                 idx_v, rows_v, x_v, *sems):
        gsem = sems[:GBUF]
        xsem = sems[GBUF:GBUF + XBUF]
        osem = sems[GBUF + XBUF:]
        wid = lax.axis_index("s") * nc + lax.axis_index("c")
        base = wid * rows_per_w
        w_per_row = L // rows_per_w

        pltpu.sync_copy(
            idx_hbm.at[wid // w_per_row,
                       pl.ds((wid % w_per_row) * rows_per_w, rows_per_w)],
            idx_v)

        def xslice(ci):
            off = base + ci * CHUNK
            return x_hbm.at[off // L, pl.ds(off % L, CHUNK)]

        def oslice(ci):
            off = base + ci * CHUNK
            return out_hbm.at[off // L, pl.ds(off % L, CHUNK)]

        def issue_g(ci, b):
            pltpu.async_copy(table_hbm.at[idx_v.at[pl.ds(ci * CHUNK, CHUNK)]],
                             rows_v.at[b], gsem[b])

        def issue_x(ci, s):
            pltpu.async_copy(xslice(ci), x_v.at[s], xsem[s])

        def wait_g(ci, b):
            pltpu.make_async_copy(
                table_hbm.at[idx_v.at[pl.ds(ci * CHUNK, CHUNK)]],
                rows_v.at[b], gsem[b]).wait()

        def wait_x(ci, s):
            pltpu.make_async_copy(xslice(ci), x_v.at[s], xsem[s]).wait()

        H = CHUNK // 2

        def ohalf(ci, h):
            off = base + ci * CHUNK + h * H
            return out_hbm.at[off // L, pl.ds(off % L, H)]

        def issue_o(ci, s, h):
            pltpu.async_copy(x_v.at[s, pl.ds(h * H, H)], ohalf(ci, h), osem[s])

        def wait_o(ci, s):
            for h in range(2):
                pltpu.make_async_copy(
                    x_v.at[s, pl.ds(h * H, H)], ohalf(ci, h), osem[s]).wait()

        for ci in range(min(XBUF, nchunk)):
            issue_x(ci, ci % XBUF)
        for ci in range(min(GBUF, nchunk)):
            issue_g(ci, ci % GBUF)

        def group(it, _):
            for k in range(XBUF):
                ci = it * XBUF + k
                s = k
                b = k % GBUF
                wait_g(ci, b)
                wait_x(ci, s)

                def add_row(r, _):
                    for j in range(E // LANES):
                        sl = pl.ds(j * LANES, LANES)
                        plsc.addupdate(x_v.at[s, r, sl], rows_v[b, r, sl])
                    return 0

                lax.fori_loop(0, H, add_row, 0)
                issue_o(ci, s, 0)
                lax.fori_loop(H, CHUNK, add_row, 0)
                issue_o(ci, s, 1)

                @pl.when(ci + GBUF < nchunk)
                def _():
                    issue_g(ci + GBUF, (k + GBUF) % GBUF)

                @pl.when(jnp.logical_and(ci >= 1, ci + XBUF - 1 < nchunk))
                def _():
                    wait_o(ci - 1, (k + XBUF - 1) % XBUF)
                    issue_x(ci + XBUF - 1, (k + XBUF - 1) % XBUF)

            return 0

        lax.fori_loop(0, nchunk // XBUF, group, 0)

        for ci in range(max(0, nchunk - XBUF), nchunk):
            wait_o(ci, ci % XBUF)

    return sc_embed


def kernel(x, variable_seq, table):
    B, L, _ = x.shape
    N = B * L
    info = plsc.get_sparse_core_info()
    num_workers = info.num_cores * info.num_subcores
    rows_per_w = N // num_workers
    nchunk = rows_per_w // CHUNK

    sc = _make_sc_kernel(B, L, num_workers, rows_per_w, nchunk)
    return sc(table, variable_seq.astype(jnp.int32), x)


# speedup vs baseline: 1.0341x; 1.0341x over previous
import functools

import jax
import jax.numpy as jnp
from jax import lax
from jax.experimental import pallas as pl
from jax.experimental.pallas import tpu as pltpu
from jax.experimental.pallas import tpu_sc as plsc

E = 768
LANES = 16
CHUNK = 16


def _make_sc_kernel(B, L, num_workers, rows_per_w, nchunk):
    mesh = plsc.VectorSubcoreMesh(core_axis_name="c", subcore_axis_name="s")
    info = plsc.get_sparse_core_info()
    nc = info.num_cores

    XBUF = 4
    GBUF = 2

    @functools.partial(
        pl.kernel,
        mesh=mesh,
        out_type=jax.ShapeDtypeStruct((B, L, E), jnp.float32),
        scratch_types=[
            pltpu.VMEM((rows_per_w,), jnp.int32),
            pltpu.VMEM((GBUF, CHUNK, E), jnp.float32),
            pltpu.VMEM((XBUF, CHUNK, E), jnp.float32),
        ] + [pltpu.SemaphoreType.DMA] * (GBUF + 2 * XBUF),
    )
    def sc_embed(table_hbm, idx_hbm, x_hbm, out_hbm,
                 idx_v, rows_v, x_v, *sems):
        gsem = sems[:GBUF]
        xsem = sems[GBUF:GBUF + XBUF]
        osem = sems[GBUF + XBUF:]
        wid = lax.axis_index("s") * nc + lax.axis_index("c")
        base = wid * rows_per_w
        w_per_row = L // rows_per_w

        pltpu.sync_copy(
            idx_hbm.at[wid // w_per_row,
                       pl.ds((wid % w_per_row) * rows_per_w, rows_per_w)],
            idx_v)

        def xslice(ci):
            off = base + ci * CHUNK
            return x_hbm.at[off // L, pl.ds(off % L, CHUNK)]

        def oslice(ci):
            off = base + ci * CHUNK
            return out_hbm.at[off // L, pl.ds(off % L, CHUNK)]

        def issue_g(ci, b):
            pltpu.async_copy(table_hbm.at[idx_v.at[pl.ds(ci * CHUNK, CHUNK)]],
                             rows_v.at[b], gsem[b])

        def issue_x(ci, s):
            pltpu.async_copy(xslice(ci), x_v.at[s], xsem[s])

        def wait_g(ci, b):
            pltpu.make_async_copy(
                table_hbm.at[idx_v.at[pl.ds(ci * CHUNK, CHUNK)]],
                rows_v.at[b], gsem[b]).wait()

        def wait_x(ci, s):
            pltpu.make_async_copy(xslice(ci), x_v.at[s], xsem[s]).wait()

        def issue_o(ci, s):
            pltpu.async_copy(x_v.at[s], oslice(ci), osem[s])

        def wait_o(ci, s):
            pltpu.make_async_copy(x_v.at[s], oslice(ci), osem[s]).wait()

        for ci in range(min(XBUF, nchunk)):
            issue_x(ci, ci % XBUF)
        for ci in range(min(GBUF, nchunk)):
            issue_g(ci, ci % GBUF)

        def group(it, _):
            for k in range(XBUF):
                ci = it * XBUF + k
                s = k
                b = k % GBUF
                wait_g(ci, b)
                wait_x(ci, s)

                @pl.when(jnp.logical_and(ci >= 1, ci + XBUF - 1 < nchunk))
                def _():
                    wait_o(ci - 1, (k + XBUF - 1) % XBUF)
                    issue_x(ci + XBUF - 1, (k + XBUF - 1) % XBUF)

                def add_row(r, _):
                    for j in range(E // LANES):
                        sl = pl.ds(j * LANES, LANES)
                        plsc.addupdate(x_v.at[s, r, sl], rows_v[b, r, sl])
                    return 0

                lax.fori_loop(0, CHUNK, add_row, 0)

                issue_o(ci, s)

                @pl.when(ci + GBUF < nchunk)
                def _():
                    issue_g(ci + GBUF, (k + GBUF) % GBUF)

            return 0

        lax.fori_loop(0, nchunk // XBUF, group, 0)

        for ci in range(max(0, nchunk - XBUF), nchunk):
            wait_o(ci, ci % XBUF)

    return sc_embed


def kernel(x, variable_seq, table):
    B, L, _ = x.shape
    N = B * L
    info = plsc.get_sparse_core_info()
    num_workers = info.num_cores * info.num_subcores
    rows_per_w = N // num_workers
    nchunk = rows_per_w // CHUNK

    sc = _make_sc_kernel(B, L, num_workers, rows_per_w, nchunk)
    return sc(table, variable_seq.astype(jnp.int32), x)
